# Initial kernel scaffold; baseline (speedup 1.0000x reference)
import jax
import jax.numpy as jnp
from jax.experimental import pallas as pl


def kernel(predictions_0, anchors_0, boxes, labels):
    raise NotImplementedError("write your pallas kernel here")


# speedup vs baseline: 10.4791x; 10.4791x over previous
import jax
import jax.numpy as jnp
from jax.experimental import pallas as pl
from jax.experimental.pallas import tpu as pltpu

_B, _A, _H, _W, _C, _M = 8, 6, 128, 128, 3, 32
_RATIO = 3.0
_CH = 32


def _smooth_l1(d):
    ad = jnp.abs(d)
    return jnp.where(ad < 1.0, 0.5 * d * d, ad - 0.5)


def _body(cx_ref, cy_ref, sc_ref, boxes_ref, labels_ref, pred_ref,
          out_ref, obj_scr, neg_scr):
    b = pl.program_id(0)

    pos_cnt = jnp.float32(0.0)
    avail = jnp.float32(0.0)
    obj_pos_sum = jnp.float32(0.0)
    ce_sum = jnp.float32(0.0)
    loc_sum = jnp.float32(0.0)

    cx = cx_ref[...]
    cyc = cy_ref[...]

    for a in range(_A):
        s = sc_ref[a]
        half = s * 0.5
        x1 = cx - half
        x2 = cx + half
        y1 = cyc - half
        y2 = cyc + half
        dx = x2 - x1
        dy = y2 - y1

        for h0 in range(0, _H, _CH):
            y1c = y1[h0:h0 + _CH, :]
            y2c = y2[h0:h0 + _CH, :]
            cyc_c = cyc[h0:h0 + _CH, :]
            area_a = dy[h0:h0 + _CH, :] * dx

            def match_step(j, carry, y1c=y1c, y2c=y2c, area_a=area_a,
                           x1=x1, x2=x2):
                biou, blbl, bx1, by1, bx2, by2 = carry
                gx1 = boxes_ref[b, j * 4 + 0]
                gy1 = boxes_ref[b, j * 4 + 1]
                gx2 = boxes_ref[b, j * 4 + 2]
                gy2 = boxes_ref[b, j * 4 + 3]
                lj = labels_ref[b, j]
                wx = jnp.maximum(jnp.minimum(x2, gx2) - jnp.maximum(x1, gx1), 0.0)
                wy = jnp.maximum(jnp.minimum(y2c, gy2) - jnp.maximum(y1c, gy1), 0.0)
                inter = wy * wx
                area_g = (gx2 - gx1) * (gy2 - gy1)
                denom = area_a + area_g
                denom = denom - inter
                denom = denom + 1e-9
                iou = inter / denom
                win = iou > biou
                biou = jnp.where(win, iou, biou)
                blbl = jnp.where(win, lj, blbl)
                bx1 = jnp.where(win, gx1, bx1)
                by1 = jnp.where(win, gy1, by1)
                bx2 = jnp.where(win, gx2, bx2)
                by2 = jnp.where(win, gy2, by2)
                return biou, blbl, bx1, by1, bx2, by2

            init = (jnp.full((_CH, _W), -1.0, jnp.float32),
                    jnp.zeros((_CH, _W), jnp.int32),
                    jnp.zeros((_CH, _W), jnp.float32),
                    jnp.zeros((_CH, _W), jnp.float32),
                    jnp.zeros((_CH, _W), jnp.float32),
                    jnp.zeros((_CH, _W), jnp.float32))
            biou, blbl, bx1, by1, bx2, by2 = jax.lax.fori_loop(
                0, _M, match_step, init)

            pos = biou >= 0.5
            neg = biou < 0.4
            posf = pos.astype(jnp.float32)
            negf = neg.astype(jnp.float32)
            pos_cnt = pos_cnt + jnp.sum(posf)
            avail = avail + jnp.sum(negf)

            x = pred_ref[0, a * 8 + 4, h0:h0 + _CH, :]
            obj_all = (jnp.maximum(x, 0.0) - x * posf
                       + jnp.log1p(jnp.exp(-jnp.abs(x))))
            r0 = a * _H + h0
            obj_scr[r0:r0 + _CH, :] = obj_all
            neg_scr[r0:r0 + _CH, :] = negf
            obj_pos_sum = obj_pos_sum + jnp.sum(jnp.where(pos, obj_all, 0.0))

            c0 = pred_ref[0, a * 8 + 5, h0:h0 + _CH, :]
            c1 = pred_ref[0, a * 8 + 6, h0:h0 + _CH, :]
            c2 = pred_ref[0, a * 8 + 7, h0:h0 + _CH, :]
            m = jnp.maximum(jnp.maximum(c0, c1), c2)
            lse = m + jnp.log(jnp.exp(c0 - m) + jnp.exp(c1 - m)
                              + jnp.exp(c2 - m))
            ct = jnp.maximum(blbl, 0)
            logit_t = jnp.where(ct == 0, c0, jnp.where(ct == 1, c1, c2))
            ce = lse - logit_t
            ce_sum = ce_sum + jnp.sum(jnp.where(pos, ce, 0.0))

            gcx = (bx1 + bx2) / 2
            gcy = (by1 + by2) / 2
            gw = jnp.maximum(bx2 - bx1, 1e-6)
            gh = jnp.maximum(by2 - by1, 1e-6)
            t0 = (gcx - cx) / s
            t1 = (gcy - cyc_c) / s
            t2 = jnp.log(gw / s)
            t3 = jnp.log(gh / s)
            sl = (_smooth_l1(pred_ref[0, a * 8 + 0, h0:h0 + _CH, :] - t0)
                  + _smooth_l1(pred_ref[0, a * 8 + 1, h0:h0 + _CH, :] - t1)
                  + _smooth_l1(pred_ref[0, a * 8 + 2, h0:h0 + _CH, :] - t2)
                  + _smooth_l1(pred_ref[0, a * 8 + 3, h0:h0 + _CH, :] - t3))
            loc_sum = loc_sum + jnp.sum(jnp.where(pos, sl, 0.0))

    kf = jnp.where(pos_cnt == 0.0,
                   jnp.minimum(100.0, avail),
                   jnp.minimum(_RATIO * pos_cnt, avail))

    negm = neg_scr[...]
    obj_v = obj_scr[...]
    bits = jax.lax.bitcast_convert_type(obj_v, jnp.int32)

    def bit_step(i, prefix):
        cand = prefix | jnp.left_shift(jnp.int32(1), 30 - i)
        cnt = jnp.sum(jnp.where((bits >= cand) & (negm > 0.0), 1.0, 0.0))
        return jnp.where(cnt >= kf, cand, prefix)

    thr = jax.lax.fori_loop(0, 31, bit_step, jnp.int32(0))
    selm = (bits >= thr) & (negm > 0.0)
    cnt_sel = jnp.sum(jnp.where(selm, 1.0, 0.0))
    sel_sum = jnp.sum(jnp.where(selm, obj_v, 0.0))

    cnt_m = pos_cnt + cnt_sel
    obj_t = jnp.where(cnt_m > 0.0,
                      (obj_pos_sum + sel_sum) / jnp.maximum(cnt_m, 1.0), 0.0)
    cls_t = jnp.where(pos_cnt > 0.0,
                      ce_sum / jnp.maximum(pos_cnt, 1.0), 0.0)
    loc_t = jnp.where(pos_cnt > 0.0,
                      loc_sum / (jnp.maximum(pos_cnt, 1.0) * 4.0), 0.0)

    lane = jax.lax.broadcasted_iota(jnp.int32, (1, _W), 1)
    row = jnp.where(lane == 0, obj_t,
                    jnp.where(lane == 1, cls_t,
                              jnp.where(lane == 2, loc_t, 0.0)))
    out_ref[...] = row[None]


def kernel(predictions_0, anchors_0, boxes, labels):
    anc4 = anchors_0.reshape(_H, _W, _A, 4)
    cx = anc4[0, :, 0, 0].reshape(1, _W)
    cy = anc4[:, 0, 0, 1].reshape(_H, 1)
    scales = anc4[0, 0, :, 2]
    boxes_flat = boxes.reshape(_B, _M * 4)
    labels32 = labels.astype(jnp.int32)

    per_image = pl.pallas_call(
        _body,
        grid=(_B,),
        in_specs=[
            pl.BlockSpec((1, _W), lambda b: (0, 0)),
            pl.BlockSpec((_H, 1), lambda b: (0, 0)),
            pl.BlockSpec(memory_space=pltpu.SMEM),
            pl.BlockSpec(memory_space=pltpu.SMEM),
            pl.BlockSpec(memory_space=pltpu.SMEM),
            pl.BlockSpec((1, (5 + _C) * _A, _H, _W), lambda b: (b, 0, 0, 0)),
        ],
        out_specs=pl.BlockSpec((1, 1, _W), lambda b: (b, 0, 0)),
        out_shape=jax.ShapeDtypeStruct((_B, 1, _W), jnp.float32),
        scratch_shapes=[
            pltpu.VMEM((_A * _H, _W), jnp.float32),
            pltpu.VMEM((_A * _H, _W), jnp.float32),
        ],
    )(cx, cy, scales, boxes_flat, labels32, predictions_0)

    lo = jnp.sum(per_image[:, 0, 0]) / _B
    lc = jnp.sum(per_image[:, 0, 1]) / _B
    ll = jnp.sum(per_image[:, 0, 2]) / _B
    return jnp.stack([lo, lc, ll, lo + lc + ll])


# speedup vs baseline: 37.8914x; 3.6159x over previous
import jax
import jax.numpy as jnp
from jax.experimental import pallas as pl
from jax.experimental.pallas import tpu as pltpu

_B, _A, _H, _W, _C, _M = 8, 6, 128, 128, 3, 32
_RATIO = 3.0
_CH = 32


def _smooth_l1(d):
    ad = jnp.abs(d)
    return jnp.where(ad < 1.0, 0.5 * d * d, ad - 0.5)


def _body(cx_ref, cy_ref, sc_ref, boxes_ref, labels_ref, pred_ref,
          out_ref, obj_scr, nbits_scr):
    b = pl.program_id(0)

    pos_cnt = jnp.float32(0.0)
    avail = jnp.float32(0.0)
    obj_pos_sum = jnp.float32(0.0)
    ce_sum = jnp.float32(0.0)
    loc_sum = jnp.float32(0.0)

    cx = cx_ref[...]
    cyc = cy_ref[...]

    for a in range(_A):
        s = sc_ref[a]
        half = s * 0.5
        x1 = cx - half
        x2 = cx + half
        y1 = cyc - half
        y2 = cyc + half
        dx = x2 - x1
        dy = y2 - y1

        for h0 in range(0, _H, _CH):
            y1c = y1[h0:h0 + _CH, :]
            y2c = y2[h0:h0 + _CH, :]
            cyc_c = cyc[h0:h0 + _CH, :]
            area_a = dy[h0:h0 + _CH, :] * dx

            binter = jnp.full((_CH, _W), -1.0, jnp.float32)
            bden = jnp.ones((_CH, _W), jnp.float32)
            blbl = jnp.zeros((_CH, _W), jnp.int32)
            bgcx = jnp.zeros((_CH, _W), jnp.float32)
            bgcy = jnp.zeros((_CH, _W), jnp.float32)
            bgw = jnp.ones((_CH, _W), jnp.float32)
            bgh = jnp.ones((_CH, _W), jnp.float32)

            for j in range(_M):
                gx1 = boxes_ref[b, j * 4 + 0]
                gy1 = boxes_ref[b, j * 4 + 1]
                gx2 = boxes_ref[b, j * 4 + 2]
                gy2 = boxes_ref[b, j * 4 + 3]
                lj = labels_ref[b, j]
                wx = jnp.maximum(jnp.minimum(x2, gx2) - jnp.maximum(x1, gx1), 0.0)
                wy = jnp.maximum(jnp.minimum(y2c, gy2) - jnp.maximum(y1c, gy1), 0.0)
                inter = wy * wx
                area_g = (gx2 - gx1) * (gy2 - gy1)
                denom = area_a + area_g
                denom = denom - inter
                denom = denom + 1e-9
                win = inter * bden > binter * denom
                binter = jnp.where(win, inter, binter)
                bden = jnp.where(win, denom, bden)
                blbl = jnp.where(win, lj, blbl)
                bgcx = jnp.where(win, (gx1 + gx2) / 2, bgcx)
                bgcy = jnp.where(win, (gy1 + gy2) / 2, bgcy)
                bgw = jnp.where(win, jnp.maximum(gx2 - gx1, 1e-6), bgw)
                bgh = jnp.where(win, jnp.maximum(gy2 - gy1, 1e-6), bgh)

            biou = binter / bden
            pos = biou >= 0.5
            neg = biou < 0.4
            posf = pos.astype(jnp.float32)
            pos_cnt = pos_cnt + jnp.sum(posf)
            avail = avail + jnp.sum(neg.astype(jnp.float32))

            x = pred_ref[0, a * 8 + 4, h0:h0 + _CH, :]
            obj_all = (jnp.maximum(x, 0.0) - x * posf
                       + jnp.log1p(jnp.exp(-jnp.abs(x))))
            r0 = a * _H + h0
            obj_scr[r0:r0 + _CH, :] = obj_all
            obits = jax.lax.bitcast_convert_type(obj_all, jnp.int32)
            nbits_scr[r0:r0 + _CH, :] = jnp.where(neg, obits, jnp.int32(-1))
            obj_pos_sum = obj_pos_sum + jnp.sum(jnp.where(pos, obj_all, 0.0))

            c0 = pred_ref[0, a * 8 + 5, h0:h0 + _CH, :]
            c1 = pred_ref[0, a * 8 + 6, h0:h0 + _CH, :]
            c2 = pred_ref[0, a * 8 + 7, h0:h0 + _CH, :]
            m = jnp.maximum(jnp.maximum(c0, c1), c2)
            lse = m + jnp.log(jnp.exp(c0 - m) + jnp.exp(c1 - m)
                              + jnp.exp(c2 - m))
            ct = jnp.maximum(blbl, 0)
            logit_t = jnp.where(ct == 0, c0, jnp.where(ct == 1, c1, c2))
            ce = lse - logit_t
            ce_sum = ce_sum + jnp.sum(jnp.where(pos, ce, 0.0))

            t0 = (bgcx - cx) / s
            t1 = (bgcy - cyc_c) / s
            t2 = jnp.log(bgw / s)
            t3 = jnp.log(bgh / s)
            sl = (_smooth_l1(pred_ref[0, a * 8 + 0, h0:h0 + _CH, :] - t0)
                  + _smooth_l1(pred_ref[0, a * 8 + 1, h0:h0 + _CH, :] - t1)
                  + _smooth_l1(pred_ref[0, a * 8 + 2, h0:h0 + _CH, :] - t2)
                  + _smooth_l1(pred_ref[0, a * 8 + 3, h0:h0 + _CH, :] - t3))
            loc_sum = loc_sum + jnp.sum(jnp.where(pos, sl, 0.0))

    kf = jnp.where(pos_cnt == 0.0,
                   jnp.minimum(100.0, avail),
                   jnp.minimum(_RATIO * pos_cnt, avail))

    nbits = nbits_scr[...]

    def bit_step(i, prefix):
        cand = prefix | jnp.left_shift(jnp.int32(1), 30 - i)
        cnt = jnp.sum((nbits >= cand).astype(jnp.float32))
        return jnp.where(cnt >= kf, cand, prefix)

    thr = jax.lax.fori_loop(0, 31, bit_step, jnp.int32(0))
    selm = nbits >= thr
    cnt_sel = jnp.sum(selm.astype(jnp.float32))
    sel_sum = jnp.sum(jnp.where(selm, obj_scr[...], 0.0))

    cnt_m = pos_cnt + cnt_sel
    obj_t = jnp.where(cnt_m > 0.0,
                      (obj_pos_sum + sel_sum) / jnp.maximum(cnt_m, 1.0), 0.0)
    cls_t = jnp.where(pos_cnt > 0.0,
                      ce_sum / jnp.maximum(pos_cnt, 1.0), 0.0)
    loc_t = jnp.where(pos_cnt > 0.0,
                      loc_sum / (jnp.maximum(pos_cnt, 1.0) * 4.0), 0.0)

    lane = jax.lax.broadcasted_iota(jnp.int32, (1, _W), 1)
    row = jnp.where(lane == 0, obj_t,
                    jnp.where(lane == 1, cls_t,
                              jnp.where(lane == 2, loc_t, 0.0)))
    out_ref[...] = row[None]


def kernel(predictions_0, anchors_0, boxes, labels):
    anc4 = anchors_0.reshape(_H, _W, _A, 4)
    cx = anc4[0, :, 0, 0].reshape(1, _W)
    cy = anc4[:, 0, 0, 1].reshape(_H, 1)
    scales = anc4[0, 0, :, 2]
    boxes_flat = boxes.reshape(_B, _M * 4)
    labels32 = labels.astype(jnp.int32)

    per_image = pl.pallas_call(
        _body,
        grid=(_B,),
        in_specs=[
            pl.BlockSpec((1, _W), lambda b: (0, 0)),
            pl.BlockSpec((_H, 1), lambda b: (0, 0)),
            pl.BlockSpec(memory_space=pltpu.SMEM),
            pl.BlockSpec(memory_space=pltpu.SMEM),
            pl.BlockSpec(memory_space=pltpu.SMEM),
            pl.BlockSpec((1, (5 + _C) * _A, _H, _W), lambda b: (b, 0, 0, 0)),
        ],
        out_specs=pl.BlockSpec((1, 1, _W), lambda b: (b, 0, 0)),
        out_shape=jax.ShapeDtypeStruct((_B, 1, _W), jnp.float32),
        scratch_shapes=[
            pltpu.VMEM((_A * _H, _W), jnp.float32),
            pltpu.VMEM((_A * _H, _W), jnp.int32),
        ],
    )(cx, cy, scales, boxes_flat, labels32, predictions_0)

    lo = jnp.sum(per_image[:, 0, 0]) / _B
    lc = jnp.sum(per_image[:, 0, 1]) / _B
    ll = jnp.sum(per_image[:, 0, 2]) / _B
    return jnp.stack([lo, lc, ll, lo + lc + ll])
